# baseline (speedup 1.0000x reference)
import functools

import jax
import jax.numpy as jnp
from jax import lax
from jax.experimental import pallas as pl
from jax.experimental.pallas import tpu as pltpu
from jax.experimental.pallas import tpu_sc as plsc

H = 4
SELU_SCALE = 1.0507009873554805
SELU_ALPHA = 1.6732632423543772

NC, NS = 2, 16
NW = NC * NS
G = 16


def _weights_body(d_ref, w_ref):
    d = d_ref[...]
    m = jnp.max(d)
    c = -4.0 / (m * m)
    d2 = d * d
    ws = []
    for h in range(H):
        wh = jnp.exp((c * (h + 1)) * d2)
        ws.append(jnp.where(wh < 1e-8, 0.0, wh))
    w_ref[...] = jnp.concatenate(ws, axis=1)


def _weights(d_p):
    npad, k = d_p.shape
    return pl.pallas_call(
        _weights_body,
        out_shape=jax.ShapeDtypeStruct((npad, H * k), jnp.float32),
    )(d_p)


def _matmul_body(a_ref, w4_ref, b4_ref, o_ref):
    y = jnp.dot(a_ref[...], w4_ref[...], preferred_element_type=jnp.float32)
    y = y + b4_ref[...]
    o_ref[...] = SELU_SCALE * jnp.where(y > 0.0, y, SELU_ALPHA * (jnp.exp(y) - 1.0))


def _matmul(agg2, w4, b4):
    npad, kred = agg2.shape
    ncol = w4.shape[1]
    blk = 1024
    return pl.pallas_call(
        _matmul_body,
        grid=(npad // blk,),
        in_specs=[
            pl.BlockSpec((blk, kred), lambda i: (i, 0)),
            pl.BlockSpec((kred, ncol), lambda i: (0, 0)),
            pl.BlockSpec((1, ncol), lambda i: (0, 0)),
        ],
        out_specs=pl.BlockSpec((blk, ncol), lambda i: (i, 0)),
        out_shape=jax.ShapeDtypeStruct((npad, ncol), jnp.float32),
    )(agg2, w4, b4.reshape(1, ncol))


@functools.lru_cache(maxsize=None)
def _make_sc_gather(npad, n_f, k_nn):
    gk = G * k_nn
    chunk = 32
    nsub = gk // chunk
    nj = n_f // 16
    row_f = H * n_f
    wk = H * k_nn
    npt_sum = npad // NS
    npt0 = (int(npt_sum * 1.0) // G) * G
    npt1 = npt_sum - npt0
    mesh = plsc.VectorSubcoreMesh(core_axis_name="c", subcore_axis_name="s")

    @functools.partial(
        pl.kernel,
        out_type=jax.ShapeDtypeStruct((npad * row_f,), jnp.float32),
        mesh=mesh,
        scratch_types=[
            pltpu.VMEM((npt0 * k_nn,), jnp.int32),
            pltpu.VMEM((npt0 * wk,), jnp.float32),
            pltpu.VMEM((2, gk, 128), jnp.float32),
            pltpu.VMEM((2 * G * row_f,), jnp.float32),
            pltpu.SemaphoreType.DMA,
            pltpu.SemaphoreType.DMA,
            pltpu.SemaphoreType.DMA,
            pltpu.SemaphoreType.DMA,
        ],
    )
    def sc_gather(nn_hbm, w_hbm, table_hbm, agg_hbm, idx_v, w_v, rows_v, acc_v,
                  sem_g0, sem_g1, sem_o0, sem_o1):
        cid = lax.axis_index("c")
        sid = lax.axis_index("s")
        sems_g = (sem_g0, sem_g1)
        sems_o = (sem_o0, sem_o1)

        def start_gather(r, buf):
            for s in range(nsub):
                pltpu.async_copy(
                    table_hbm.at[idx_v.at[pl.ds(r * gk + s * chunk, chunk)]],
                    rows_v.at[buf, pl.ds(s * chunk, chunk), :],
                    sems_g[buf])

        def wait_gather(buf):
            for s in range(nsub):
                pltpu.make_async_copy(
                    table_hbm.at[idx_v.at[pl.ds(s * chunk, chunk)]],
                    rows_v.at[buf, pl.ds(s * chunk, chunk), :],
                    sems_g[buf]).wait()

        def combine(r, buf):
            def node_body(g, carry):
                rowbase = g * k_nn
                node = r * G + g
                wv = [w_v[pl.ds(node * wk + h * k_nn, k_nn)] for h in range(H)]
                accs = [jnp.zeros((16,), jnp.float32) for _ in range(H * nj)]
                for k in range(k_nn):
                    rv = [rows_v[buf, rowbase + k, pl.ds(16 * j, 16)]
                          for j in range(nj)]
                    kidx = jnp.full((16,), k, jnp.int32)
                    for h in range(H):
                        wkh = wv[h][kidx]
                        for j in range(nj):
                            accs[h * nj + j] = accs[h * nj + j] + wkh * rv[j]
                for h in range(H):
                    for j in range(nj):
                        acc_v[pl.ds(buf * G * row_f + g * row_f
                                    + h * n_f + j * 16, 16)] = accs[h * nj + j]
                return 0

            lax.fori_loop(0, G, node_body, 0)

        def pipeline(base, npt):
            rounds = npt // G

            def out_slot(r):
                return pl.ds((base + r * G) * row_f, G * row_f)

            def start_out(r, buf):
                pltpu.async_copy(acc_v.at[pl.ds(buf * G * row_f, G * row_f)],
                                 agg_hbm.at[out_slot(r)], sems_o[buf])

            def wait_out(r, buf):
                pltpu.make_async_copy(
                    acc_v.at[pl.ds(buf * G * row_f, G * row_f)],
                    agg_hbm.at[out_slot(r)], sems_o[buf]).wait()

            pltpu.sync_copy(nn_hbm.at[pl.ds(base * k_nn, npt * k_nn)],
                            idx_v.at[pl.ds(0, npt * k_nn)])
            pltpu.sync_copy(w_hbm.at[pl.ds(base * wk, npt * wk)],
                            w_v.at[pl.ds(0, npt * wk)])
            start_gather(0, 0)

            def one_round(r, buf):
                @pl.when(r + 1 < rounds)
                def _():
                    start_gather(r + 1, 1 - buf)

                wait_gather(buf)

                @pl.when(r >= 2)
                def _():
                    wait_out(r - 2, buf)

                combine(r, buf)
                start_out(r, buf)

            def round_pair(r2, carry):
                for buf in range(2):
                    one_round(2 * r2 + buf, buf)
                return 0

            lax.fori_loop(0, rounds // 2, round_pair, 0)
            if rounds % 2:
                one_round(rounds - 1, 0)
                wait_out(rounds - 2, 1)
                wait_out(rounds - 1, 0)
            else:
                wait_out(rounds - 2, 0)
                wait_out(rounds - 1, 1)

        @pl.when(cid == 0)
        def _():
            pipeline(sid * npt0, npt0)

        if npt1 > 0:
            @pl.when(cid == 1)
            def _():
                pipeline(NS * npt0 + sid * npt1, npt1)

    return sc_gather


def kernel(x, nearest_nodes, nearest_dists, W, b):
    B, T, N, C = x.shape
    k_nn = nearest_nodes.shape[1]
    O = W.shape[0]
    n_f = B * T * C

    align = NW * G
    npad = ((N + align - 1) // align) * align

    xt = jnp.transpose(x, (2, 0, 1, 3)).reshape(N, n_f)
    xt = jnp.pad(xt, ((0, 0), (0, 128 - n_f)))
    nn_p = jnp.pad(nearest_nodes, ((0, npad - N), (0, 0))).reshape(npad * k_nn)
    d_p = jnp.pad(nearest_dists, ((0, npad - N), (0, 0)))

    wr = W.reshape(O, T, H)
    w4 = jnp.einsum(
        'oth,bB,cC->hbtcBCo',
        wr,
        jnp.eye(B, dtype=jnp.float32),
        jnp.eye(C, dtype=jnp.float32),
    ).reshape(H * n_f, B * C * O)
    b4 = jnp.tile(b, B * C)

    wts = _weights(d_p).reshape(npad * H * k_nn)
    agg = _make_sc_gather(npad, n_f, k_nn)(nn_p, wts, xt)
    y2 = _matmul(agg.reshape(npad, H * n_f), w4, b4)

    y = y2[:N].reshape(N, B, C, O).transpose(1, 3, 0, 2)
    return jnp.concatenate([x, y], axis=1)


# speedup vs baseline: 1.1925x; 1.1925x over previous
import functools

import jax
import jax.numpy as jnp
from jax import lax
from jax.experimental import pallas as pl
from jax.experimental.pallas import tpu as pltpu
from jax.experimental.pallas import tpu_sc as plsc

H = 4
SELU_SCALE = 1.0507009873554805
SELU_ALPHA = 1.6732632423543772

NC, NS = 2, 16
NW = NC * NS
G = 16


def _weights_body(d_ref, w_ref):
    d = d_ref[...]
    m = jnp.max(d)
    c = -4.0 / (m * m)
    d2 = d * d
    ws = []
    for h in range(H):
        wh = jnp.exp((c * (h + 1)) * d2)
        ws.append(jnp.where(wh < 1e-8, 0.0, wh))
    w_ref[...] = jnp.concatenate(ws, axis=1)


def _weights(d_p):
    npad, k = d_p.shape
    return pl.pallas_call(
        _weights_body,
        out_shape=jax.ShapeDtypeStruct((npad, H * k), jnp.float32),
    )(d_p)


def _matmul_body(a_ref, w4_ref, b4_ref, o_ref):
    y = jnp.dot(a_ref[...], w4_ref[...], preferred_element_type=jnp.float32)
    y = y + b4_ref[...]
    o_ref[...] = SELU_SCALE * jnp.where(y > 0.0, y, SELU_ALPHA * (jnp.exp(y) - 1.0))


def _matmul(agg2, w4, b4):
    npad, kred = agg2.shape
    ncol = w4.shape[1]
    blk = 1024
    return pl.pallas_call(
        _matmul_body,
        grid=(npad // blk,),
        in_specs=[
            pl.BlockSpec((blk, kred), lambda i: (i, 0)),
            pl.BlockSpec((kred, ncol), lambda i: (0, 0)),
            pl.BlockSpec((1, ncol), lambda i: (0, 0)),
        ],
        out_specs=pl.BlockSpec((blk, ncol), lambda i: (i, 0)),
        out_shape=jax.ShapeDtypeStruct((npad, ncol), jnp.float32),
    )(agg2, w4, b4.reshape(1, ncol))


@functools.lru_cache(maxsize=None)
def _make_sc_gather(npad, n_f, k_nn):
    gk = G * k_nn
    chunk = 32
    nsub = gk // chunk
    nj = n_f // 16
    row_f = H * n_f
    wk = H * k_nn
    npt_sum = npad // NS
    npt0 = (int(npt_sum * 0.9) // G) * G
    npt1 = npt_sum - npt0
    mesh = plsc.VectorSubcoreMesh(core_axis_name="c", subcore_axis_name="s")

    @functools.partial(
        pl.kernel,
        out_type=jax.ShapeDtypeStruct((npad * row_f,), jnp.float32),
        mesh=mesh,
        scratch_types=[
            pltpu.VMEM((npt0 * k_nn,), jnp.int32),
            pltpu.VMEM((npt0 * wk,), jnp.float32),
            pltpu.VMEM((2, gk, 128), jnp.float32),
            pltpu.VMEM((2 * G * row_f,), jnp.float32),
            pltpu.SemaphoreType.DMA,
            pltpu.SemaphoreType.DMA,
            pltpu.SemaphoreType.DMA,
            pltpu.SemaphoreType.DMA,
        ],
    )
    def sc_gather(nn_hbm, w_hbm, table_hbm, agg_hbm, idx_v, w_v, rows_v, acc_v,
                  sem_g0, sem_g1, sem_o0, sem_o1):
        cid = lax.axis_index("c")
        sid = lax.axis_index("s")
        sems_g = (sem_g0, sem_g1)
        sems_o = (sem_o0, sem_o1)

        def start_gather(r, buf):
            for s in range(nsub):
                pltpu.async_copy(
                    table_hbm.at[idx_v.at[pl.ds(r * gk + s * chunk, chunk)]],
                    rows_v.at[buf, pl.ds(s * chunk, chunk), :],
                    sems_g[buf])

        def wait_gather(buf):
            for s in range(nsub):
                pltpu.make_async_copy(
                    table_hbm.at[idx_v.at[pl.ds(s * chunk, chunk)]],
                    rows_v.at[buf, pl.ds(s * chunk, chunk), :],
                    sems_g[buf]).wait()

        def combine(r, buf):
            def node_body(g, carry):
                rowbase = g * k_nn
                node = r * G + g
                wv = [w_v[pl.ds(node * wk + h * k_nn, k_nn)] for h in range(H)]
                accs = [jnp.zeros((16,), jnp.float32) for _ in range(H * nj)]
                for k in range(k_nn):
                    rv = [rows_v[buf, rowbase + k, pl.ds(16 * j, 16)]
                          for j in range(nj)]
                    kidx = jnp.full((16,), k, jnp.int32)
                    for h in range(H):
                        wkh = wv[h][kidx]
                        for j in range(nj):
                            accs[h * nj + j] = accs[h * nj + j] + wkh * rv[j]
                for h in range(H):
                    for j in range(nj):
                        acc_v[pl.ds(buf * G * row_f + g * row_f
                                    + h * n_f + j * 16, 16)] = accs[h * nj + j]
                return 0

            lax.fori_loop(0, G, node_body, 0)

        def pipeline(base, npt):
            rounds = npt // G

            def out_slot(r):
                return pl.ds((base + r * G) * row_f, G * row_f)

            def start_out(r, buf):
                pltpu.async_copy(acc_v.at[pl.ds(buf * G * row_f, G * row_f)],
                                 agg_hbm.at[out_slot(r)], sems_o[buf])

            def wait_out(r, buf):
                pltpu.make_async_copy(
                    acc_v.at[pl.ds(buf * G * row_f, G * row_f)],
                    agg_hbm.at[out_slot(r)], sems_o[buf]).wait()

            pltpu.sync_copy(nn_hbm.at[pl.ds(base * k_nn, npt * k_nn)],
                            idx_v.at[pl.ds(0, npt * k_nn)])
            pltpu.sync_copy(w_hbm.at[pl.ds(base * wk, npt * wk)],
                            w_v.at[pl.ds(0, npt * wk)])
            start_gather(0, 0)

            def one_round(r, buf):
                @pl.when(r + 1 < rounds)
                def _():
                    start_gather(r + 1, 1 - buf)

                wait_gather(buf)

                @pl.when(r >= 2)
                def _():
                    wait_out(r - 2, buf)

                combine(r, buf)
                start_out(r, buf)

            def round_pair(r2, carry):
                for buf in range(2):
                    one_round(2 * r2 + buf, buf)
                return 0

            lax.fori_loop(0, rounds // 2, round_pair, 0)
            if rounds % 2:
                one_round(rounds - 1, 0)
                wait_out(rounds - 2, 1)
                wait_out(rounds - 1, 0)
            else:
                wait_out(rounds - 2, 0)
                wait_out(rounds - 1, 1)

        @pl.when(cid == 0)
        def _():
            pipeline(sid * npt0, npt0)

        if npt1 > 0:
            @pl.when(cid == 1)
            def _():
                pipeline(NS * npt0 + sid * npt1, npt1)

    return sc_gather


def kernel(x, nearest_nodes, nearest_dists, W, b):
    B, T, N, C = x.shape
    k_nn = nearest_nodes.shape[1]
    O = W.shape[0]
    n_f = B * T * C

    align = NW * G
    npad = ((N + align - 1) // align) * align

    xt = jnp.transpose(x, (2, 0, 1, 3)).reshape(N, n_f)
    xt = jnp.pad(xt, ((0, 0), (0, 128 - n_f)))
    nn_p = jnp.pad(nearest_nodes, ((0, npad - N), (0, 0))).reshape(npad * k_nn)
    d_p = jnp.pad(nearest_dists, ((0, npad - N), (0, 0)))

    wr = W.reshape(O, T, H)
    w4 = jnp.einsum(
        'oth,bB,cC->hbtcBCo',
        wr,
        jnp.eye(B, dtype=jnp.float32),
        jnp.eye(C, dtype=jnp.float32),
    ).reshape(H * n_f, B * C * O)
    b4 = jnp.tile(b, B * C)

    wts = _weights(d_p).reshape(npad * H * k_nn)
    agg = _make_sc_gather(npad, n_f, k_nn)(nn_p, wts, xt)
    y2 = _matmul(agg.reshape(npad, H * n_f), w4, b4)

    y = y2[:N].reshape(N, B, C, O).transpose(1, 3, 0, 2)
    return jnp.concatenate([x, y], axis=1)
